# baseline (speedup 1.0000x reference)
import functools

import jax
import jax.numpy as jnp
from jax import lax
from jax.experimental import pallas as pl
from jax.experimental.pallas import tpu as pltpu
from jax.experimental.pallas import tpu_sc as plsc

_BATCH = 16384
_HIST = 50
_DIM = 64
_B = _BATCH * _HIST

_NC = 2
_NS = 16
_NW = _NC * _NS
_B_PER_W = _B // _NW

_CHUNK = 128
_CHUNKS_PER_W = _B_PER_W // _CHUNK
_GROUP = 4
_ROWS_PER_GROUP = _GROUP * _CHUNK
_N_GROUPS = _CHUNKS_PER_W // _GROUP


def _emb_body(idx_hbm, table_hbm, out_hbm, idx_v,
              rows0, rows1, rows2, gsem0, gsem1, gsem2, wsem0, wsem1, wsem2):
    cid = lax.axis_index("c")
    sid = lax.axis_index("s")
    wid = sid * _NC + cid
    cbase = wid * _CHUNKS_PER_W
    rbase = wid * _B_PER_W

    bufs = (rows0, rows1, rows2)
    gsems = (gsem0, gsem1, gsem2)
    wsems = (wsem0, wsem1, wsem2)

    pltpu.sync_copy(idx_hbm.at[pl.ds(cbase, _CHUNKS_PER_W)], idx_v)

    def ig(g, b):
        for j in range(_GROUP):
            pltpu.async_copy(
                table_hbm.at[idx_v.at[g * _GROUP + j]],
                bufs[b].at[pl.ds(j * _CHUNK, _CHUNK)],
                gsems[b],
            )

    def wg(b):
        pltpu.make_async_copy(
            table_hbm.at[pl.ds(0, _ROWS_PER_GROUP)], bufs[b], gsems[b]
        ).wait()

    def iw(g, b):
        pltpu.async_copy(
            bufs[b],
            out_hbm.at[pl.ds(rbase + g * _ROWS_PER_GROUP, _ROWS_PER_GROUP)],
            wsems[b],
        )

    def ww(g, b):
        pltpu.make_async_copy(
            bufs[b],
            out_hbm.at[pl.ds(rbase + g * _ROWS_PER_GROUP, _ROWS_PER_GROUP)],
            wsems[b],
        ).wait()

    ig(0, 0)
    ig(1, 1)
    wg(0); iw(0, 0); ig(2, 2)

    @pl.loop(1, _N_GROUPS - 4, step=3)
    def _(g0):
        for k in range(3):
            g = g0 + k
            b = (1 + k) % 3
            wg(b); iw(g, b); ww(g - 1, (b + 2) % 3); ig(g + 2, (b + 2) % 3)

    n = _N_GROUPS
    wg(1); iw(n - 4, 1); ww(n - 5, 0); ig(n - 2, 0)
    wg(2); iw(n - 3, 2); ww(n - 4, 1); ig(n - 1, 1)
    wg(0); iw(n - 2, 0); ww(n - 3, 2)
    wg(1); iw(n - 1, 1); ww(n - 2, 0)
    ww(n - 1, 1)


@jax.jit
def kernel(token_ids, weights):
    idx = token_ids.astype(jnp.int32).reshape(_NW * _CHUNKS_PER_W, _CHUNK)
    run = pl.kernel(
        _emb_body,
        out_type=jax.ShapeDtypeStruct((_B, _DIM), jnp.float32),
        mesh=plsc.VectorSubcoreMesh(core_axis_name="c", subcore_axis_name="s"),
        scratch_types=[
            pltpu.VMEM((_CHUNKS_PER_W, _CHUNK), jnp.int32),
            pltpu.VMEM((_ROWS_PER_GROUP, _DIM), jnp.float32),
            pltpu.VMEM((_ROWS_PER_GROUP, _DIM), jnp.float32),
            pltpu.VMEM((_ROWS_PER_GROUP, _DIM), jnp.float32),
            pltpu.SemaphoreType.DMA,
            pltpu.SemaphoreType.DMA,
            pltpu.SemaphoreType.DMA,
            pltpu.SemaphoreType.DMA,
            pltpu.SemaphoreType.DMA,
            pltpu.SemaphoreType.DMA,
        ],
        compiler_params=pltpu.CompilerParams(use_tc_tiling_on_sc=False),
    )
    out = run(idx, weights)
    return out.reshape(_BATCH, _HIST, _DIM)
